# baseline (speedup 1.0000x reference)
import functools

import jax
import jax.numpy as jnp
from jax.experimental import pallas as pl

_DEF = jax.lax.Precision.DEFAULT

NQ = 6
NB_CODE = 1024
CODE_DIM = 512


def _mm_body(relu_in, relu_out, has_res, *refs):
    if has_res:
        x_ref, w_ref, b_ref, res_ref, out_ref = refs
    else:
        x_ref, w_ref, b_ref, out_ref = refs
    x = x_ref[...]
    if relu_in:
        x = jnp.maximum(x, 0.0)
    acc = jax.lax.dot_general(
        x, w_ref[...], dimension_numbers=(((1,), (0,)), ((), ())),
        precision=_DEF, preferred_element_type=jnp.float32)
    acc = acc + b_ref[...]
    if has_res:
        acc = acc + res_ref[...]
    if relu_out:
        acc = jnp.maximum(acc, 0.0)
    out_ref[...] = acc


def _mm(big, wbig, b, relu_in=False, relu_out=False, res=None):
    R, K = big.shape
    Cout = wbig.shape[1]
    RB = 1024 if R % 1024 == 0 else R
    grid = (R // RB,)
    in_specs = [pl.BlockSpec((RB, K), lambda i: (i, 0)),
                pl.BlockSpec((K, Cout), lambda i: (0, 0)),
                pl.BlockSpec((1, Cout), lambda i: (0, 0))]
    args = [big, wbig, b.reshape(1, Cout)]
    if res is not None:
        in_specs.append(pl.BlockSpec((RB, Cout), lambda i: (i, 0)))
        args.append(res.reshape(R, Cout))
    return pl.pallas_call(
        functools.partial(_mm_body, relu_in, relu_out, res is not None),
        grid=grid,
        in_specs=in_specs,
        out_specs=pl.BlockSpec((RB, Cout), lambda i: (i, 0)),
        out_shape=jax.ShapeDtypeStruct((R, Cout), jnp.float32),
    )(*args)


def _conv(h, p, pad=0, dil=1, stride=1, relu_in=False, relu_out=False,
          res=None, use_xla=False):
    N, T, Cin = h.shape
    w = p["w"]
    Cout, _, k = w.shape
    if use_xla:
        hx = h
        if relu_in:
            hx = jnp.maximum(hx, 0.0)
        out = jax.lax.conv_general_dilated(
            hx.transpose(0, 2, 1), w, (stride,), [(pad, pad)],
            rhs_dilation=(dil,), dimension_numbers=("NCH", "OIH", "NCH"))
        out = (out + p["b"][None, :, None]).transpose(0, 2, 1)
        if res is not None:
            out = out + res
        if relu_out:
            out = jnp.maximum(out, 0.0)
        return out
    hp = jnp.pad(h, ((0, 0), (pad, pad), (0, 0))) if pad else h
    if stride == 1:
        T_out = T + 2 * pad - dil * (k - 1)
        taps = [hp[:, j * dil: j * dil + T_out, :] for j in range(k)]
    else:
        T_out = (T + 2 * pad - k) // stride + 1
        taps = [hp[:, j::stride, :][:, :T_out, :] for j in range(k)]
    big = jnp.concatenate(taps, axis=2).reshape(N * T_out, k * Cin)
    wbig = jnp.transpose(w, (2, 1, 0)).reshape(k * Cin, Cout)
    out = _mm(big, wbig, p["b"], relu_in=relu_in, relu_out=relu_out, res=res)
    return out.reshape(N, T_out, Cout)


def _resnet(h, blocks, dils, c1_xla=False):
    for p, d in zip(blocks, dils):
        t = _conv(h, p["c1"], pad=d, dil=d, relu_in=True, use_xla=c1_xla)
        h = _conv(t, p["c2"], relu_in=True, res=h)
    return h


def _rvq_stage_body(res_ref, a_ref, cbT_ref, cbh_ref, cbm_ref, cbl_ref,
                    csq_ref, resout_ref, stats_ref):
    r = res_ref[...]
    Rr = r.shape[0]
    s = jax.lax.dot_general(
        r, cbT_ref[...], dimension_numbers=(((1,), (0,)), ((), ())),
        precision=_DEF, preferred_element_type=jnp.float32)
    d = a_ref[...] - 2.0 * s + csq_ref[...]
    m = jnp.min(d, axis=1, keepdims=True)
    lane_iota = jax.lax.broadcasted_iota(jnp.int32, (Rr, NB_CODE), 1)
    idx = jnp.min(jnp.where(d == m, lane_iota, NB_CODE), axis=1,
                  keepdims=True)
    onehot = (lane_iota == idx).astype(jnp.bfloat16)
    qs = []
    for part in (cbh_ref, cbm_ref, cbl_ref):
        qs.append(jax.lax.dot_general(
            onehot, part[...], dimension_numbers=(((1,), (0,)), ((), ())),
            precision=_DEF, preferred_element_type=jnp.float32))
    q = (qs[0] + qs[1]) + qs[2]
    resout_ref[...] = r - q
    closs = jnp.mean((r - q) ** 2)
    counts = jnp.sum(onehot.astype(jnp.float32), axis=0, keepdims=True)
    pr = counts / jnp.float32(Rr)
    perp = jnp.exp(-jnp.sum(pr * jnp.log(pr + 1e-10)))
    r8 = jax.lax.broadcasted_iota(jnp.int32, (8, 128), 0)
    c8 = jax.lax.broadcasted_iota(jnp.int32, (8, 128), 1)
    stats_ref[...] = (jnp.where((r8 == 0) & (c8 == 0), closs, 0.0)
                      + jnp.where((r8 == 0) & (c8 == 1), perp, 0.0))


def _rvq(flat, codebooks):
    R, C = flat.shape
    residual = flat
    closs = jnp.float32(0.0)
    perps = []
    for i in range(NQ):
        cb = codebooks[i]
        cb_h = cb.astype(jnp.bfloat16)
        r1 = cb - cb_h.astype(jnp.float32)
        cb_m = r1.astype(jnp.bfloat16)
        cb_l = (r1 - cb_m.astype(jnp.float32)).astype(jnp.bfloat16)
        a = jnp.sum(residual ** 2, axis=1, keepdims=True)
        csq = jnp.sum(cb ** 2, axis=1)[None, :]
        residual, stats = pl.pallas_call(
            _rvq_stage_body,
            out_shape=(jax.ShapeDtypeStruct((R, C), jnp.float32),
                       jax.ShapeDtypeStruct((8, 128), jnp.float32)),
        )(residual, a, cb.T, cb_h, cb_m, cb_l, csq)
        closs = closs + stats[0, 0]
        perps.append(stats[0, 1])
    qout = flat - residual
    return qout, closs, jnp.mean(jnp.stack(perps))


def kernel(x, params):
    enc, dec, cbs = params["enc"], params["dec"], params["codebooks"]
    h = _conv(x, enc["cin"], pad=1, relu_out=True, use_xla=True)
    for bi, blk in enumerate(enc["down"]):
        h = _conv(h, blk["conv"], pad=1, stride=2)
        h = _resnet(h, blk["res"], (1, 3, 9), c1_xla=(bi == 0))
    xe = _conv(h, enc["cout"], pad=1)
    N = xe.shape[0]
    flat = xe.reshape(N * xe.shape[1], CODE_DIM)
    qout, closs, perp = _rvq(flat, cbs)
    h = _conv(qout.reshape(N, -1, CODE_DIM), dec["cin"], pad=1, relu_out=True)
    for blk in dec["up"]:
        h = _resnet(h, blk["res"], (9, 3, 1))
        h = jnp.repeat(h, 2, axis=1)
        h = _conv(h, blk["conv"], pad=1)
    h = _conv(h, dec["out1"], pad=1, relu_out=True)
    y = _conv(h, dec["out2"], pad=1)
    return y, closs, perp
